# baseline (speedup 1.0000x reference)
import functools

import jax
import jax.numpy as jnp
from jax import lax
from jax.experimental import pallas as pl
from jax.experimental.pallas import tpu as pltpu
from jax.experimental.pallas import tpu_sc as plsc

NC = 2
NS = 16
NW = NC * NS

_CK = 64
_NBK = 3
_ZR = 208

_DCK = 40


def _sc_mesh():
    return plsc.VectorSubcoreMesh(core_axis_name="c", subcore_axis_name="s")


def _make_deg_kernel(ew, NP):
    nch = ew // _DCK
    pt = NP // NS

    @functools.partial(
        pl.kernel,
        mesh=_sc_mesh(),
        out_type=jax.ShapeDtypeStruct((NC, NP), jnp.float32),
        scratch_types=[
            pltpu.VMEM((pt,), jnp.float32),
            pltpu.VMEM((_DCK,), jnp.float32),
            pltpu.VMEM((nch, _DCK), jnp.int32),
            pltpu.VMEM_SHARED((NP,), jnp.float32),
            pltpu.SemaphoreType.DMA((4,)),
        ],
    )
    def deg_kernel(dstr_hbm, degp_hbm, zbuf, ones_v, didx, dacc, dsem):
        c = lax.axis_index("c")
        s = lax.axis_index("s")
        wid = c * NS + s

        pltpu.sync_copy(dstr_hbm.at[wid], didx)

        def _zero(i, _):
            zbuf[pl.ds(i * 16, 16)] = jnp.zeros((16,), jnp.float32)
            return 0
        lax.fori_loop(0, pt // 16, _zero, 0)
        for j in range(_DCK // 16):
            ones_v[pl.ds(j * 16, 16)] = jnp.ones((16,), jnp.float32)
        ones_v[pl.ds(_DCK - 16, 16)] = jnp.ones((16,), jnp.float32)
        pltpu.sync_copy(zbuf, dacc.at[pl.ds(s * pt, pt)])
        plsc.subcore_barrier()

        def _sc_add(ch, b):
            return pltpu.make_async_copy(
                ones_v, dacc.at[didx.at[ch]], dsem.at[b])

        def _body(g, _):
            c0 = g * 4
            for b in range(4):
                _sc_add(c0 + b, b).start(add=True)
            for b in range(4):
                _sc_add(c0 + b, b).wait()
            return 0
        lax.fori_loop(0, nch // 4, _body, 0)

        plsc.subcore_barrier()
        pltpu.sync_copy(dacc.at[pl.ds(s * pt, pt)],
                        degp_hbm.at[c, pl.ds(s * pt, pt)])

    return deg_kernel


def _make_agg_kernel(N, ew, D):
    nch = ew // _CK
    NA = N + 8
    rpt = (N // NS) // 8 * 8
    tail = N - rpt * NS
    nz = rpt // _ZR

    @functools.partial(
        pl.kernel,
        mesh=_sc_mesh(),
        out_type=jax.ShapeDtypeStruct((NC, N, D), jnp.float32),
        scratch_types=[
            pltpu.VMEM((_NBK, 2, _CK), jnp.int32),
            pltpu.VMEM((_NBK, _CK, D), jnp.float32),
            pltpu.VMEM_SHARED((NA, D), jnp.float32),
            pltpu.SemaphoreType.DMA((_NBK,)),
            pltpu.SemaphoreType.DMA((_NBK,)),
            pltpu.SemaphoreType.DMA((_NBK,)),
        ],
    )
    def agg_kernel(g_hbm, sd_hbm, zeros_hbm, part_hbm,
                   idxr, rows, acc, isem, gsem, ssem):
        c = lax.axis_index("c")
        s = lax.axis_index("s")
        wid = c * NS + s

        for k in range(nz):
            pltpu.sync_copy(zeros_hbm, acc.at[pl.ds(s * rpt + k * _ZR, _ZR)])
        if tail:
            @pl.when(s == NS - 1)
            def _zero_tail():
                pltpu.sync_copy(zeros_hbm.at[pl.ds(0, tail)],
                                acc.at[pl.ds(NS * rpt, tail)])
        plsc.subcore_barrier()

        def _idx(ch, B):
            return pltpu.make_async_copy(
                sd_hbm.at[wid, ch], idxr.at[B], isem.at[B])

        def _gather(B):
            return pltpu.make_async_copy(
                g_hbm.at[idxr.at[B, 0]], rows.at[B], gsem.at[B])

        def _scatter(B):
            return pltpu.make_async_copy(
                rows.at[B], acc.at[idxr.at[B, 1]], ssem.at[B])

        def _fire_gather(ch, B):
            _idx(ch, B).wait()
            _gather(B).start()

        def _fire_scatter(B):
            _gather(B).wait()
            _scatter(B).start(add=True)

        _idx(0, 0).start()
        _fire_gather(0, 0)
        _idx(1, 1).start()
        _fire_scatter(0)
        _fire_gather(1, 1)
        _idx(2, 2).start()
        _fire_scatter(1)

        def _step(ch, B, Bn):
            _fire_gather(ch, B)
            _scatter(Bn).wait()
            _idx(ch + 1, Bn).start()
            _fire_scatter(B)

        def _body(p, _):
            g0 = 3 * p + 2
            _step(g0, 2, 0)
            _step(g0 + 1, 0, 1)
            _step(g0 + 2, 1, 2)
            return 0
        lax.fori_loop(0, (nch - 3) // 3, _body, 0)

        _step(nch - 2, (nch - 2) % 3, (nch - 1) % 3)
        _fire_gather(nch - 1, (nch - 1) % 3)
        _scatter(nch % 3).wait()
        _fire_scatter((nch - 1) % 3)
        _scatter((nch - 2) % 3).wait()
        _scatter((nch - 1) % 3).wait()

        plsc.subcore_barrier()
        pltpu.sync_copy(acc.at[pl.ds(s * rpt, rpt)],
                        part_hbm.at[c, pl.ds(s * rpt, rpt)])
        if tail:
            @pl.when(s == NS - 1)
            def _copy_tail():
                pltpu.sync_copy(acc.at[pl.ds(NS * rpt, tail)],
                                part_hbm.at[c, pl.ds(NS * rpt, tail)])

    return agg_kernel


def _scale_mm_body(x_ref, w_ref, d0_ref, d1_ref, out_ref, dis_ref):
    d = d0_ref[...] + d1_ref[...] + 1.0
    dis = lax.rsqrt(d)
    dis_ref[...] = dis
    h = jnp.dot(x_ref[...], w_ref[...], preferred_element_type=jnp.float32)
    out_ref[...] = h * dis


def _mid_body(p_ref, g_ref, dis_ref, b_ref, w_ref, out_ref):
    t = (p_ref[0] + p_ref[1] + g_ref[...]) * dis_ref[...] + b_ref[...]
    o = jnp.maximum(t, 0.0)
    out_ref[...] = jnp.dot(
        o, w_ref[...], preferred_element_type=jnp.float32) * dis_ref[...]


def _final_body(p_ref, g_ref, dis_ref, b_ref, out_ref):
    out_ref[...] = ((p_ref[0] + p_ref[1] + g_ref[...]) * dis_ref[...]
                    + b_ref[...])


def kernel(x, edge_index, W1, b1, W2, b2):
    N, D = x.shape
    E = edge_index.shape[1]
    NP = 10240

    gran = NW * 320
    E_pad = -(-E // gran) * gran
    pad = E_pad - E
    ew = E_pad // NW
    nch = ew // _CK

    src_p = jnp.concatenate(
        [edge_index[0], (jnp.arange(pad, dtype=jnp.int32) % N)])
    dst_p = jnp.concatenate(
        [edge_index[1], N + (jnp.arange(pad, dtype=jnp.int32) % 8)])
    sd = jnp.stack([src_p.reshape(NW, nch, _CK),
                    dst_p.reshape(NW, nch, _CK)], axis=2)
    dst_r = dst_p.reshape(NW, ew // _DCK, _DCK)

    degp = _make_deg_kernel(ew, NP)(dst_r)

    BM = 1000
    grid = (N // BM,)
    row_spec = pl.BlockSpec((BM, D), lambda i: (i, 0))
    dis_spec = pl.BlockSpec((BM, 1), lambda i: (i, 0))
    w_spec = pl.BlockSpec((D, D), lambda i: (0, 0))
    b_spec = pl.BlockSpec((1, D), lambda i: (0, 0))
    p_spec = pl.BlockSpec((NC, BM, D), lambda i: (0, i, 0))
    out_sds = jax.ShapeDtypeStruct((N, D), jnp.float32)

    agg = _make_agg_kernel(N, ew, D)
    zeros_z = jnp.zeros((_ZR, D), jnp.float32)

    g1, dis = pl.pallas_call(
        _scale_mm_body, grid=grid,
        in_specs=[row_spec, w_spec, dis_spec, dis_spec],
        out_specs=[row_spec, dis_spec],
        out_shape=[out_sds, jax.ShapeDtypeStruct((N, 1), jnp.float32)],
    )(x, W1, degp[0].reshape(NP, 1), degp[1].reshape(NP, 1))

    p1 = agg(g1, sd, zeros_z)

    g2 = pl.pallas_call(
        _mid_body, grid=grid,
        in_specs=[p_spec, row_spec, dis_spec, b_spec, w_spec],
        out_specs=row_spec, out_shape=out_sds,
    )(p1, g1, dis, b1.reshape(1, D), W2)

    p2 = agg(g2, sd, zeros_z)

    out = pl.pallas_call(
        _final_body, grid=grid,
        in_specs=[p_spec, row_spec, dis_spec, b_spec],
        out_specs=row_spec, out_shape=out_sds,
    )(p2, g2, dis, b2.reshape(1, D))

    return out


# speedup vs baseline: 1.1196x; 1.1196x over previous
import functools

import jax
import jax.numpy as jnp
from jax import lax
from jax.experimental import pallas as pl
from jax.experimental.pallas import tpu as pltpu
from jax.experimental.pallas import tpu_sc as plsc

NC = 2
NS = 16
NW = NC * NS

_CK = 40
_K = 2
_NB = 2
_ZR = 208

_DCK = 40


def _sc_mesh():
    return plsc.VectorSubcoreMesh(core_axis_name="c", subcore_axis_name="s")


def _make_deg_kernel(ew, NP):
    nch = ew // _DCK
    pt = NP // NS

    @functools.partial(
        pl.kernel,
        mesh=_sc_mesh(),
        out_type=jax.ShapeDtypeStruct((NC, NP), jnp.float32),
        scratch_types=[
            pltpu.VMEM((pt,), jnp.float32),
            pltpu.VMEM((_DCK,), jnp.float32),
            pltpu.VMEM((nch, _DCK), jnp.int32),
            pltpu.VMEM_SHARED((NP,), jnp.float32),
            pltpu.SemaphoreType.DMA((4,)),
        ],
    )
    def deg_kernel(dstr_hbm, degp_hbm, zbuf, ones_v, didx, dacc, dsem):
        c = lax.axis_index("c")
        s = lax.axis_index("s")
        wid = c * NS + s

        pltpu.sync_copy(dstr_hbm.at[wid], didx)

        def _zero(i, _):
            zbuf[pl.ds(i * 16, 16)] = jnp.zeros((16,), jnp.float32)
            return 0
        lax.fori_loop(0, pt // 16, _zero, 0)
        for j in range(_DCK // 16):
            ones_v[pl.ds(j * 16, 16)] = jnp.ones((16,), jnp.float32)
        ones_v[pl.ds(_DCK - 16, 16)] = jnp.ones((16,), jnp.float32)
        pltpu.sync_copy(zbuf, dacc.at[pl.ds(s * pt, pt)])
        plsc.subcore_barrier()

        def _sc_add(ch, b):
            return pltpu.make_async_copy(
                ones_v, dacc.at[didx.at[ch]], dsem.at[b])

        def _body(g, _):
            c0 = g * 4
            for b in range(4):
                _sc_add(c0 + b, b).start(add=True)
            for b in range(4):
                _sc_add(c0 + b, b).wait()
            return 0
        lax.fori_loop(0, nch // 4, _body, 0)

        plsc.subcore_barrier()
        pltpu.sync_copy(dacc.at[pl.ds(s * pt, pt)],
                        degp_hbm.at[c, pl.ds(s * pt, pt)])

    return deg_kernel


def _make_agg_kernel(N, ew, D):
    nch = ew // _CK
    NA = N + 8
    rpt = (N // NS) // 8 * 8
    tail = N - rpt * NS
    nz = rpt // _ZR

    @functools.partial(
        pl.kernel,
        mesh=_sc_mesh(),
        out_type=jax.ShapeDtypeStruct((NC, N, D), jnp.float32),
        scratch_types=[
            pltpu.VMEM((_NB * _K, 2, _CK), jnp.int32),
            pltpu.VMEM((_NB * _K, _CK, D), jnp.float32),
            pltpu.VMEM_SHARED((NA, D), jnp.float32),
            pltpu.SemaphoreType.DMA((_NB * _K,)),
            pltpu.SemaphoreType.DMA((_NB * _K,)),
            pltpu.SemaphoreType.DMA((_NB * _K,)),
        ],
    )
    def agg_kernel(g_hbm, sd_hbm, zeros_hbm, part_hbm,
                   idxr, rows, acc, isem, gsem, ssem):
        c = lax.axis_index("c")
        s = lax.axis_index("s")
        wid = c * NS + s

        for k in range(nz):
            pltpu.sync_copy(zeros_hbm, acc.at[pl.ds(s * rpt + k * _ZR, _ZR)])
        if tail:
            @pl.when(s == NS - 1)
            def _zero_tail():
                pltpu.sync_copy(zeros_hbm.at[pl.ds(0, tail)],
                                acc.at[pl.ds(NS * rpt, tail)])
        plsc.subcore_barrier()

        def _idx(ch, b):
            return pltpu.make_async_copy(
                sd_hbm.at[wid, ch], idxr.at[b], isem.at[b])

        def _gather(b):
            return pltpu.make_async_copy(
                g_hbm.at[idxr.at[b, 0]], rows.at[b], gsem.at[b])

        def _scatter(b):
            return pltpu.make_async_copy(
                rows.at[b], acc.at[idxr.at[b, 1]], ssem.at[b])

        def _fire_idx(g, B):
            for b in range(_K):
                _idx(g * _K + b, B + b).start()

        def _fire_gather(g, B):
            for b in range(_K):
                _idx(g * _K + b, B + b).wait()
                _gather(B + b).start()

        def _fire_scatter(B):
            for b in range(_K):
                _gather(B + b).wait()
                _scatter(B + b).start(add=True)

        def _drain_scatter(B):
            for b in range(_K):
                _scatter(B + b).wait()

        ngrp = nch // _K

        _fire_idx(0, 0)
        _fire_gather(0, 0)
        _fire_idx(1, _K)
        _fire_scatter(0)

        def _step(g, B, Bo):
            _fire_gather(g, B)
            _drain_scatter(Bo)
            _fire_idx(g + 1, Bo)
            _fire_scatter(B)

        def _body(p, _):
            _step(2 * p + 1, _K, 0)
            _step(2 * p + 2, 0, _K)
            return 0
        lax.fori_loop(0, (ngrp - 2) // 2, _body, 0)

        _fire_gather(ngrp - 1, _K)
        _drain_scatter(0)
        _fire_scatter(_K)
        _drain_scatter(_K)

        plsc.subcore_barrier()
        pltpu.sync_copy(acc.at[pl.ds(s * rpt, rpt)],
                        part_hbm.at[c, pl.ds(s * rpt, rpt)])
        if tail:
            @pl.when(s == NS - 1)
            def _copy_tail():
                pltpu.sync_copy(acc.at[pl.ds(NS * rpt, tail)],
                                part_hbm.at[c, pl.ds(NS * rpt, tail)])

    return agg_kernel


def _scale_mm_body(x_ref, w_ref, d0_ref, d1_ref, out_ref, dis_ref):
    d = d0_ref[...] + d1_ref[...] + 1.0
    dis = lax.rsqrt(d)
    dis_ref[...] = dis
    h = jnp.dot(x_ref[...], w_ref[...], preferred_element_type=jnp.float32)
    out_ref[...] = h * dis


def _mid_body(p_ref, g_ref, dis_ref, b_ref, w_ref, out_ref):
    t = (p_ref[0] + p_ref[1] + g_ref[...]) * dis_ref[...] + b_ref[...]
    o = jnp.maximum(t, 0.0)
    out_ref[...] = jnp.dot(
        o, w_ref[...], preferred_element_type=jnp.float32) * dis_ref[...]


def _final_body(p_ref, g_ref, dis_ref, b_ref, out_ref):
    out_ref[...] = ((p_ref[0] + p_ref[1] + g_ref[...]) * dis_ref[...]
                    + b_ref[...])


def kernel(x, edge_index, W1, b1, W2, b2):
    N, D = x.shape
    E = edge_index.shape[1]
    NP = 10240

    gran = NW * 320
    E_pad = -(-E // gran) * gran
    pad = E_pad - E
    ew = E_pad // NW
    nch = ew // _CK

    src_p = jnp.concatenate(
        [edge_index[0], (jnp.arange(pad, dtype=jnp.int32) % N)])
    dst_p = jnp.concatenate(
        [edge_index[1], N + (jnp.arange(pad, dtype=jnp.int32) % 8)])
    sd = jnp.stack([src_p.reshape(NW, nch, _CK),
                    dst_p.reshape(NW, nch, _CK)], axis=2)
    dst_r = dst_p.reshape(NW, ew // _DCK, _DCK)

    degp = _make_deg_kernel(ew, NP)(dst_r)

    BM = 1000
    grid = (N // BM,)
    row_spec = pl.BlockSpec((BM, D), lambda i: (i, 0))
    dis_spec = pl.BlockSpec((BM, 1), lambda i: (i, 0))
    w_spec = pl.BlockSpec((D, D), lambda i: (0, 0))
    b_spec = pl.BlockSpec((1, D), lambda i: (0, 0))
    p_spec = pl.BlockSpec((NC, BM, D), lambda i: (0, i, 0))
    out_sds = jax.ShapeDtypeStruct((N, D), jnp.float32)

    agg = _make_agg_kernel(N, ew, D)
    zeros_z = jnp.zeros((_ZR, D), jnp.float32)

    g1, dis = pl.pallas_call(
        _scale_mm_body, grid=grid,
        in_specs=[row_spec, w_spec, dis_spec, dis_spec],
        out_specs=[row_spec, dis_spec],
        out_shape=[out_sds, jax.ShapeDtypeStruct((N, 1), jnp.float32)],
    )(x, W1, degp[0].reshape(NP, 1), degp[1].reshape(NP, 1))

    p1 = agg(g1, sd, zeros_z)

    g2 = pl.pallas_call(
        _mid_body, grid=grid,
        in_specs=[p_spec, row_spec, dis_spec, b_spec, w_spec],
        out_specs=row_spec, out_shape=out_sds,
    )(p1, g1, dis, b1.reshape(1, D), W2)

    p2 = agg(g2, sd, zeros_z)

    out = pl.pallas_call(
        _final_body, grid=grid,
        in_specs=[p_spec, row_spec, dis_spec, b_spec],
        out_specs=row_spec, out_shape=out_sds,
    )(p2, g2, dis, b2.reshape(1, D))

    return out


# speedup vs baseline: 1.1352x; 1.0140x over previous
import functools

import jax
import jax.numpy as jnp
from jax import lax
from jax.experimental import pallas as pl
from jax.experimental.pallas import tpu as pltpu
from jax.experimental.pallas import tpu_sc as plsc

NC = 2
NS = 16
NW = NC * NS

_CK = 40
_K = 2
_NB = 2
_ZR = 208

_DCK = 40


def _sc_mesh():
    return plsc.VectorSubcoreMesh(core_axis_name="c", subcore_axis_name="s")


def _make_deg_kernel(ew, NP):
    nch = ew // _DCK
    pt = NP // NS

    @functools.partial(
        pl.kernel,
        mesh=_sc_mesh(),
        out_type=jax.ShapeDtypeStruct((NC, NP), jnp.float32),
        scratch_types=[
            pltpu.VMEM((pt,), jnp.float32),
            pltpu.VMEM((_DCK,), jnp.float32),
            pltpu.VMEM((nch, _DCK), jnp.int32),
            pltpu.VMEM_SHARED((NP,), jnp.float32),
            pltpu.SemaphoreType.DMA((4,)),
        ],
    )
    def deg_kernel(dstr_hbm, degp_hbm, zbuf, ones_v, didx, dacc, dsem):
        c = lax.axis_index("c")
        s = lax.axis_index("s")
        wid = c * NS + s

        pltpu.sync_copy(dstr_hbm.at[wid], didx)

        def _zero(i, _):
            zbuf[pl.ds(i * 16, 16)] = jnp.zeros((16,), jnp.float32)
            return 0
        lax.fori_loop(0, pt // 16, _zero, 0)
        for j in range(_DCK // 16):
            ones_v[pl.ds(j * 16, 16)] = jnp.ones((16,), jnp.float32)
        ones_v[pl.ds(_DCK - 16, 16)] = jnp.ones((16,), jnp.float32)
        pltpu.sync_copy(zbuf, dacc.at[pl.ds(s * pt, pt)])
        plsc.subcore_barrier()

        def _sc_add(ch, b):
            return pltpu.make_async_copy(
                ones_v, dacc.at[didx.at[ch]], dsem.at[b])

        def _body(g, _):
            c0 = g * 4
            for b in range(4):
                _sc_add(c0 + b, b).start(add=True)
            for b in range(4):
                _sc_add(c0 + b, b).wait()
            return 0
        lax.fori_loop(0, nch // 4, _body, 0)

        plsc.subcore_barrier()
        pltpu.sync_copy(dacc.at[pl.ds(s * pt, pt)],
                        degp_hbm.at[c, pl.ds(s * pt, pt)])

    return deg_kernel


def _make_agg_kernel(N, ew, D):
    nch = ew // _CK
    NA = N + 8
    rpt = (N // NS) // 8 * 8
    tail = N - rpt * NS
    nz = rpt // _ZR

    @functools.partial(
        pl.kernel,
        mesh=_sc_mesh(),
        out_type=jax.ShapeDtypeStruct((NC, N, D), jnp.float32),
        scratch_types=[
            pltpu.VMEM((_NB * _K, 2, _CK), jnp.int32),
            pltpu.VMEM((_NB * _K, _CK, D), jnp.float32),
            pltpu.VMEM_SHARED((NA, D), jnp.float32),
            pltpu.SemaphoreType.DMA((_NB * _K,)),
            pltpu.SemaphoreType.DMA((_NB * _K,)),
            pltpu.SemaphoreType.DMA((_NB * _K,)),
        ],
    )
    def agg_kernel(g_hbm, sd_hbm, zeros_hbm, part_hbm,
                   idxr, rows, acc, isem, gsem, ssem):
        c = lax.axis_index("c")
        s = lax.axis_index("s")
        wid = c * NS + s

        for k in range(nz):
            pltpu.sync_copy(zeros_hbm, acc.at[pl.ds(s * rpt + k * _ZR, _ZR)])
        if tail:
            @pl.when(s == NS - 1)
            def _zero_tail():
                pltpu.sync_copy(zeros_hbm.at[pl.ds(0, tail)],
                                acc.at[pl.ds(NS * rpt, tail)])
        plsc.subcore_barrier()

        def _idx(ch, b):
            return pltpu.make_async_copy(
                sd_hbm.at[wid, ch], idxr.at[b], isem.at[b])

        def _gather(b):
            return pltpu.make_async_copy(
                g_hbm.at[idxr.at[b, 0]], rows.at[b], gsem.at[b])

        def _scatter(b):
            return pltpu.make_async_copy(
                rows.at[b], acc.at[idxr.at[b, 1]], ssem.at[b])

        def _fire_idx(g, B):
            for b in range(_K):
                _idx(g * _K + b, B + b).start()

        def _fire_gather(g, B):
            for b in range(_K):
                _idx(g * _K + b, B + b).wait()
                _gather(B + b).start()

        def _fire_scatter(B):
            for b in range(_K):
                _gather(B + b).wait()
                _scatter(B + b).start(add=True)

        def _drain_scatter(B):
            for b in range(_K):
                _scatter(B + b).wait()

        ngrp = nch // _K

        _fire_idx(0, 0)
        _fire_gather(0, 0)
        _fire_idx(1, _K)
        _fire_scatter(0)

        def _step(g, B, Bo):
            _fire_gather(g, B)
            _drain_scatter(Bo)
            _fire_idx(g + 1, Bo)
            _fire_scatter(B)

        def _body(p, _):
            _step(2 * p + 1, _K, 0)
            _step(2 * p + 2, 0, _K)
            return 0
        lax.fori_loop(0, (ngrp - 2) // 2, _body, 0)

        _fire_gather(ngrp - 1, _K)
        _drain_scatter(0)
        _fire_scatter(_K)
        _drain_scatter(_K)

        plsc.subcore_barrier()
        pltpu.sync_copy(acc.at[pl.ds(s * rpt, rpt)],
                        part_hbm.at[c, pl.ds(s * rpt, rpt)])
        if tail:
            @pl.when(s == NS - 1)
            def _copy_tail():
                pltpu.sync_copy(acc.at[pl.ds(NS * rpt, tail)],
                                part_hbm.at[c, pl.ds(NS * rpt, tail)])

    return agg_kernel


def _scale_mm_body(x_ref, w_ref, d0_ref, d1_ref, out_ref, dis_ref):
    d = d0_ref[...] + d1_ref[...] + 1.0
    dis = lax.rsqrt(d)
    dis_ref[...] = dis
    h = jnp.dot(x_ref[...], w_ref[...], preferred_element_type=jnp.float32)
    out_ref[...] = h * dis


def _mid_body(p_ref, g_ref, dis_ref, b_ref, w_ref, out_ref):
    t = (p_ref[0] + p_ref[1] + g_ref[...]) * dis_ref[...] + b_ref[...]
    o = jnp.maximum(t, 0.0)
    out_ref[...] = jnp.dot(
        o, w_ref[...], preferred_element_type=jnp.float32) * dis_ref[...]


def _final_body(p_ref, g_ref, dis_ref, b_ref, out_ref):
    out_ref[...] = ((p_ref[0] + p_ref[1] + g_ref[...]) * dis_ref[...]
                    + b_ref[...])


def kernel(x, edge_index, W1, b1, W2, b2):
    N, D = x.shape
    E = edge_index.shape[1]
    NP = 10240

    gran = NW * 320
    E_pad = -(-E // gran) * gran
    pad = E_pad - E
    ew = E_pad // NW
    nch = ew // _CK

    src_p = jnp.concatenate(
        [edge_index[0], (jnp.arange(pad, dtype=jnp.int32) % N)])
    dst_p = jnp.concatenate(
        [edge_index[1], N + (jnp.arange(pad, dtype=jnp.int32) % 8)])
    sd = jnp.stack([src_p.reshape(NW, nch, _CK),
                    dst_p.reshape(NW, nch, _CK)], axis=2)
    dst_r = dst_p.reshape(NW, ew // _DCK, _DCK)

    degp = _make_deg_kernel(ew, NP)(dst_r)

    BM = 2000
    grid = (N // BM,)
    row_spec = pl.BlockSpec((BM, D), lambda i: (i, 0))
    dis_spec = pl.BlockSpec((BM, 1), lambda i: (i, 0))
    w_spec = pl.BlockSpec((D, D), lambda i: (0, 0))
    b_spec = pl.BlockSpec((1, D), lambda i: (0, 0))
    p_spec = pl.BlockSpec((NC, BM, D), lambda i: (0, i, 0))
    out_sds = jax.ShapeDtypeStruct((N, D), jnp.float32)

    agg = _make_agg_kernel(N, ew, D)
    zeros_z = jnp.zeros((_ZR, D), jnp.float32)

    g1, dis = pl.pallas_call(
        _scale_mm_body, grid=grid,
        in_specs=[row_spec, w_spec, dis_spec, dis_spec],
        out_specs=[row_spec, dis_spec],
        out_shape=[out_sds, jax.ShapeDtypeStruct((N, 1), jnp.float32)],
    )(x, W1, degp[0].reshape(NP, 1), degp[1].reshape(NP, 1))

    p1 = agg(g1, sd, zeros_z)

    g2 = pl.pallas_call(
        _mid_body, grid=grid,
        in_specs=[p_spec, row_spec, dis_spec, b_spec, w_spec],
        out_specs=row_spec, out_shape=out_sds,
    )(p1, g1, dis, b1.reshape(1, D), W2)

    p2 = agg(g2, sd, zeros_z)

    out = pl.pallas_call(
        _final_body, grid=grid,
        in_specs=[p_spec, row_spec, dis_spec, b_spec],
        out_specs=row_spec, out_shape=out_sds,
    )(p2, g2, dis, b2.reshape(1, D))

    return out
